# Initial kernel scaffold; baseline (speedup 1.0000x reference)
import jax
import jax.numpy as jnp
from jax.experimental import pallas as pl


def kernel(x, gw_i, gb_i, w1_i, b1_i, w2_i, b2_i, gw_s, gb_s, w1_s, b1_s, w2_s, b2_s):
    raise NotImplementedError("write your pallas kernel here")


# speedup vs baseline: 1.5372x; 1.5372x over previous
import functools

import jax
import jax.numpy as jnp
from jax.experimental import pallas as pl
from jax.experimental.pallas import tpu as pltpu

T = 2048
H = 768
EH = 768
EI = 32
ES = 4
K = 16


def _router_kernel(x_ref, gwi_ref, gbi_ref, gws_ref, gbs_ref,
                   gi_ref, gs_ref, aux_ref):
    x = x_ref[...]
    li = jnp.dot(x, gwi_ref[...], preferred_element_type=jnp.float32)
    li = li + gbi_ref[...]
    p = jax.nn.softmax(li, axis=-1)

    iota = jax.lax.broadcasted_iota(jnp.int32, (T, EI), 1)
    rem = p
    sel = jnp.zeros((T, EI), dtype=jnp.bool_)
    for _ in range(K):
        m = jnp.max(rem, axis=-1, keepdims=True)
        ismax = rem == m
        first = jnp.min(jnp.where(ismax, iota, EI), axis=-1, keepdims=True)
        pick = iota == first
        sel = jnp.logical_or(sel, pick)
        rem = jnp.where(pick, -jnp.inf, rem)

    pv = jnp.where(sel, p, 0.0)
    gates_i = pv / jnp.sum(pv, axis=-1, keepdims=True)
    gi_ref[...] = gates_i

    density = jnp.mean(sel.astype(jnp.float32), axis=0)
    mean_prob = jnp.mean(p, axis=0)
    aux = jnp.float32(EI) * jnp.sum(density * mean_prob)
    aux_ref[...] = jnp.reshape(aux, (1, 1))

    ls = jnp.dot(x, gws_ref[...], preferred_element_type=jnp.float32)
    ls = ls + gbs_ref[...]
    gs_ref[...] = jax.nn.softmax(ls, axis=-1)


def _expert_kernel(num_e, x_ref, w1_ref, b1_ref, w2_ref, b2_ref, g_ref,
                   out_ref):
    e = pl.program_id(0)
    onehot = (jax.lax.broadcasted_iota(jnp.int32, (num_e, 1), 0) == e
              ).astype(jnp.float32)
    g = jnp.dot(g_ref[...], onehot,
                preferred_element_type=jnp.float32)
    h = jnp.dot(x_ref[...], w1_ref[0], preferred_element_type=jnp.float32)
    h = jax.nn.gelu(h + b1_ref[0])
    y = jnp.dot(h * g, w2_ref[0], preferred_element_type=jnp.float32)
    contrib = y + g * b2_ref[0]

    @pl.when(e == 0)
    def _():
        out_ref[...] = contrib

    @pl.when(e > 0)
    def _():
        out_ref[...] += contrib


def _run_experts(x, w1, b1, w2, b2, gates, num_e):
    return pl.pallas_call(
        functools.partial(_expert_kernel, num_e),
        grid=(num_e,),
        in_specs=[
            pl.BlockSpec((T, H), lambda e: (0, 0)),
            pl.BlockSpec((1, H, EH), lambda e: (e, 0, 0)),
            pl.BlockSpec((1, 1, EH), lambda e: (e, 0, 0)),
            pl.BlockSpec((1, EH, H), lambda e: (e, 0, 0)),
            pl.BlockSpec((1, 1, H), lambda e: (e, 0, 0)),
            pl.BlockSpec((T, num_e), lambda e: (0, 0)),
        ],
        out_specs=pl.BlockSpec((T, H), lambda e: (0, 0)),
        out_shape=jax.ShapeDtypeStruct((T, H), jnp.float32),
    )(x, w1, b1, w2, b2, gates)


@jax.jit
def kernel(x, gw_i, gb_i, w1_i, b1_i, w2_i, b2_i,
           gw_s, gb_s, w1_s, b1_s, w2_s, b2_s):
    gates_i, gates_s, aux = pl.pallas_call(
        _router_kernel,
        out_shape=(
            jax.ShapeDtypeStruct((T, EI), jnp.float32),
            jax.ShapeDtypeStruct((T, ES), jnp.float32),
            jax.ShapeDtypeStruct((1, 1), jnp.float32),
        ),
    )(x, gw_i, gb_i.reshape(1, EI), gw_s, gb_s.reshape(1, ES))

    out_i = _run_experts(x, w1_i, b1_i.reshape(EI, 1, EH),
                         w2_i, b2_i.reshape(EI, 1, H), gates_i, EI)
    out_s = _run_experts(x, w1_s, b1_s.reshape(ES, 1, EH),
                         w2_s, b2_s.reshape(ES, 1, H), gates_s, ES)
    return out_i + out_s, aux[0, 0]
